# Initial kernel scaffold; baseline (speedup 1.0000x reference)
import jax
import jax.numpy as jnp
from jax.experimental import pallas as pl


def kernel(nodes, neigh_idx, features, W):
    raise NotImplementedError("write your pallas kernel here")


# speedup vs baseline: 6.1072x; 6.1072x over previous
import functools

import jax
import jax.numpy as jnp
from jax import lax
from jax.experimental import pallas as pl
from jax.experimental.pallas import tpu as pltpu
from jax.experimental.pallas import tpu_sc as plsc

B = 16384
K = 32
FEAT = 128
EMB = 128
LANES = 16
NC, NS = 2, 16
NW = NC * NS
SEEDS_PER_W = B // NW
IDX_PER_W = SEEDS_PER_W * K
CHUNK_IDX = 128
SEEDS_PER_CHUNK = CHUNK_IDX // K
NCHUNK = IDX_PER_W // CHUNK_IDX
NSEG = FEAT // LANES


def _sc_gather_mean(idx, features):
    mesh = plsc.VectorSubcoreMesh(core_axis_name="c", subcore_axis_name="s")

    @functools.partial(
        pl.kernel,
        out_type=jax.ShapeDtypeStruct((B, FEAT), jnp.float32),
        mesh=mesh,
        scratch_types=[
            pltpu.VMEM((NCHUNK, CHUNK_IDX), jnp.int32),
            pltpu.VMEM((CHUNK_IDX, FEAT), jnp.float32),
            pltpu.VMEM((SEEDS_PER_W, FEAT), jnp.float32),
            pltpu.SemaphoreType.DMA,
        ],
    )
    def body(idx_hbm, feat_hbm, out_hbm, idx_v, buf_v, out_v, sem):
        wid = lax.axis_index("s") * NC + lax.axis_index("c")
        pltpu.sync_copy(idx_hbm.at[wid], idx_v)

        def do_chunk(j, carry):
            pltpu.async_copy(feat_hbm.at[idx_v.at[j]], buf_v, sem).wait()
            for n in range(SEEDS_PER_CHUNK):
                def kstep(kk, accs):
                    row = n * K + kk
                    return tuple(
                        accs[d] + buf_v[row, pl.ds(d * LANES, LANES)]
                        for d in range(NSEG)
                    )
                accs = lax.fori_loop(
                    0, K, kstep,
                    tuple(jnp.zeros((LANES,), jnp.float32) for _ in range(NSEG)),
                )
                for d in range(NSEG):
                    out_v[j * SEEDS_PER_CHUNK + n, pl.ds(d * LANES, LANES)] = (
                        accs[d] * (1.0 / K)
                    )
            return carry

        lax.fori_loop(0, NCHUNK, do_chunk, 0)
        pltpu.sync_copy(out_v, out_hbm.at[pl.ds(wid * SEEDS_PER_W, SEEDS_PER_W)])

    return body(idx, features)


def _project(combined, W):
    def mm(x_ref, w_ref, o_ref):
        o_ref[...] = lax.dot_general(
            x_ref[...], w_ref[...], (((1,), (1,)), ((), ())),
            preferred_element_type=jnp.float32,
        )

    return pl.pallas_call(
        mm,
        grid=(B // 1024,),
        in_specs=[
            pl.BlockSpec((1024, FEAT), lambda i: (i, 0)),
            pl.BlockSpec((FEAT, FEAT), lambda i: (0, 0)),
        ],
        out_specs=pl.BlockSpec((1024, EMB), lambda i: (i, 0)),
        out_shape=jax.ShapeDtypeStruct((B, EMB), jnp.float32),
    )(combined, W)


def kernel(nodes, neigh_idx, features, W):
    del nodes
    idx = neigh_idx.astype(jnp.int32).reshape(NW, NCHUNK, CHUNK_IDX)
    combined = _sc_gather_mean(idx, features)
    return _project(combined, W)


# speedup vs baseline: 9.3848x; 1.5367x over previous
import functools

import jax
import jax.numpy as jnp
from jax import lax
from jax.experimental import pallas as pl
from jax.experimental.pallas import tpu as pltpu
from jax.experimental.pallas import tpu_sc as plsc

B = 16384
K = 32
FEAT = 128
EMB = 128
LANES = 16
NC, NS = 2, 16
NW = NC * NS
SEEDS_PER_W = B // NW
IDX_PER_W = SEEDS_PER_W * K
CHUNK_IDX = 128
SEEDS_PER_CHUNK = CHUNK_IDX // K
NCHUNK = IDX_PER_W // CHUNK_IDX
NSEG = FEAT // LANES


def _sc_gather_mean(idx, features):
    mesh = plsc.VectorSubcoreMesh(core_axis_name="c", subcore_axis_name="s")

    @functools.partial(
        pl.kernel,
        out_type=jax.ShapeDtypeStruct((B, FEAT), jnp.float32),
        mesh=mesh,
        scratch_types=[
            pltpu.VMEM((NCHUNK, CHUNK_IDX), jnp.int32),
            pltpu.VMEM((CHUNK_IDX, FEAT), jnp.float32),
            pltpu.VMEM((CHUNK_IDX, FEAT), jnp.float32),
            pltpu.VMEM((SEEDS_PER_W, FEAT), jnp.float32),
            pltpu.SemaphoreType.DMA,
            pltpu.SemaphoreType.DMA,
        ],
    )
    def body(idx_hbm, feat_hbm, out_hbm, idx_v, buf0, buf1, out_v, sem0, sem1):
        wid = lax.axis_index("s") * NC + lax.axis_index("c")
        bufs, sems = (buf0, buf1), (sem0, sem1)
        pltpu.sync_copy(idx_hbm.at[wid], idx_v)

        def accum_chunk(j, buf):
            def node_step(n, carry):
                for d in range(NSEG):
                    col = pl.ds(d * LANES, LANES)
                    parts = [buf[n * K + k, col] for k in range(4)]
                    for k in range(4, K):
                        parts[k % 4] = parts[k % 4] + buf[n * K + k, col]
                    s = (parts[0] + parts[1]) + (parts[2] + parts[3])
                    out_v[j * SEEDS_PER_CHUNK + n, col] = s * (1.0 / K)
                return carry

            lax.fori_loop(0, SEEDS_PER_CHUNK, node_step, 0)

        for b in range(2):
            pltpu.async_copy(feat_hbm.at[idx_v.at[b]], bufs[b], sems[b])

        def step(i, carry):
            for b in range(2):
                j = 2 * i + b
                pltpu.make_async_copy(
                    feat_hbm.at[idx_v.at[j]], bufs[b], sems[b]).wait()
                accum_chunk(j, bufs[b])
                pltpu.async_copy(
                    feat_hbm.at[idx_v.at[j + 2]], bufs[b], sems[b])
            return carry

        lax.fori_loop(0, (NCHUNK - 2) // 2, step, 0)
        for b in range(2):
            j = NCHUNK - 2 + b
            pltpu.make_async_copy(
                feat_hbm.at[idx_v.at[j]], bufs[b], sems[b]).wait()
            accum_chunk(j, bufs[b])

        pltpu.sync_copy(out_v, out_hbm.at[pl.ds(wid * SEEDS_PER_W, SEEDS_PER_W)])

    return body(idx, features)


def _project(combined, W):
    def mm(x_ref, w_ref, o_ref):
        o_ref[...] = lax.dot_general(
            x_ref[...], w_ref[...], (((1,), (1,)), ((), ())),
            preferred_element_type=jnp.float32,
        )

    return pl.pallas_call(
        mm,
        grid=(B // 1024,),
        in_specs=[
            pl.BlockSpec((1024, FEAT), lambda i: (i, 0)),
            pl.BlockSpec((FEAT, FEAT), lambda i: (0, 0)),
        ],
        out_specs=pl.BlockSpec((1024, EMB), lambda i: (i, 0)),
        out_shape=jax.ShapeDtypeStruct((B, EMB), jnp.float32),
    )(combined, W)


def kernel(nodes, neigh_idx, features, W):
    del nodes
    idx = neigh_idx.astype(jnp.int32).reshape(NW, NCHUNK, CHUNK_IDX)
    combined = _sc_gather_mean(idx, features)
    return _project(combined, W)


# speedup vs baseline: 13.6538x; 1.4549x over previous
import functools

import jax
import jax.numpy as jnp
from jax import lax
from jax.experimental import pallas as pl
from jax.experimental.pallas import tpu as pltpu
from jax.experimental.pallas import tpu_sc as plsc

B = 16384
K = 32
FEAT = 128
EMB = 128
NC, NS = 2, 16
NW = NC * NS
SEEDS_PER_W = B // NW
CHUNK = 128
NCH = SEEDS_PER_W // CHUNK


def _sc_gather_sum(idx, features):
    mesh = plsc.VectorSubcoreMesh(core_axis_name="c", subcore_axis_name="s")

    @functools.partial(
        pl.kernel,
        out_type=jax.ShapeDtypeStruct((B, FEAT), jnp.float32),
        mesh=mesh,
        scratch_types=[
            pltpu.VMEM((K, SEEDS_PER_W), jnp.int32),
            pltpu.VMEM((CHUNK, FEAT), jnp.float32),
            pltpu.VMEM((CHUNK, FEAT), jnp.float32),
            pltpu.SemaphoreType.DMA,
            pltpu.SemaphoreType.DMA,
            pltpu.SemaphoreType.DMA,
            pltpu.SemaphoreType.DMA,
            pltpu.SemaphoreType.DMA,
            pltpu.SemaphoreType.DMA,
        ],
    )
    def body(idx_hbm, feat_hbm, out_hbm, idx_v,
             buf0, buf1, ow0, ow1, ad0, ad1, wb0, wb1):
        wid = lax.axis_index("s") * NC + lax.axis_index("c")
        bufs, ows, ads, wbs = (buf0, buf1), (ow0, ow1), (ad0, ad1), (wb0, wb1)
        pltpu.sync_copy(idx_hbm.at[wid], idx_v)

        def issue_overwrite(j):
            b = j % 2
            pltpu.async_copy(
                feat_hbm.at[idx_v.at[0, pl.ds(j * CHUNK, CHUNK)]], bufs[b], ows[b])

        def start_adds(j):
            b = j % 2
            pltpu.make_async_copy(
                feat_hbm.at[idx_v.at[0, pl.ds(j * CHUNK, CHUNK)]],
                bufs[b], ows[b]).wait()
            for k in range(1, K):
                pltpu.async_copy(
                    feat_hbm.at[idx_v.at[k, pl.ds(j * CHUNK, CHUNK)]],
                    bufs[b], ads[b], add=True)

        def writeback(j):
            b = j % 2
            dst = out_hbm.at[pl.ds(wid * SEEDS_PER_W + j * CHUNK, CHUNK)]
            for k in range(1, K):
                pltpu.make_async_copy(
                    feat_hbm.at[idx_v.at[k, pl.ds(j * CHUNK, CHUNK)]],
                    bufs[b], ads[b]).wait()
            pltpu.async_copy(bufs[b], dst, wbs[b])

        def wait_writeback(j):
            b = j % 2
            dst = out_hbm.at[pl.ds(wid * SEEDS_PER_W + j * CHUNK, CHUNK)]
            pltpu.make_async_copy(bufs[b], dst, wbs[b]).wait()

        issue_overwrite(0)
        start_adds(0)
        issue_overwrite(1)
        start_adds(1)
        writeback(0)
        wait_writeback(0)
        issue_overwrite(2)
        start_adds(2)
        writeback(1)
        wait_writeback(1)
        issue_overwrite(3)
        start_adds(3)
        writeback(2)
        writeback(3)
        wait_writeback(2)
        wait_writeback(3)

    return body(idx, features)


def _project(combined, Wt):
    def mm(x_ref, w_ref, o_ref):
        o_ref[...] = lax.dot_general(
            x_ref[...], w_ref[...], (((1,), (0,)), ((), ())),
            preferred_element_type=jnp.float32,
        )

    return pl.pallas_call(
        mm,
        grid=(B // 1024,),
        in_specs=[
            pl.BlockSpec((1024, FEAT), lambda i: (i, 0)),
            pl.BlockSpec((FEAT, EMB), lambda i: (0, 0)),
        ],
        out_specs=pl.BlockSpec((1024, EMB), lambda i: (i, 0)),
        out_shape=jax.ShapeDtypeStruct((B, EMB), jnp.float32),
    )(combined, Wt)


def kernel(nodes, neigh_idx, features, W):
    del nodes
    idx = (neigh_idx.astype(jnp.int32)
           .reshape(NW, SEEDS_PER_W, K)
           .transpose(0, 2, 1))
    sums = _sc_gather_sum(idx, features)
    Wt = W.T * (1.0 / K)
    return _project(sums, Wt)
